# Initial kernel scaffold; baseline (speedup 1.0000x reference)
import jax
import jax.numpy as jnp
from jax.experimental import pallas as pl


def kernel(x, edge_index, adj_values, W, b):
    raise NotImplementedError("write your pallas kernel here")


# speedup vs baseline: 3.1391x; 3.1391x over previous
import functools

import jax
import jax.numpy as jnp
from jax import lax
from jax.experimental import pallas as pl
from jax.experimental.pallas import tpu as pltpu
from jax.experimental.pallas import tpu_sc as plsc

_NC = 2
_NS = 16
_L = 16

_GROUP = 128
_GB = 4


def _matmul_body(x_ref, w_ref, b_ref, o_ref):
  d_half = o_ref.shape[2]
  h = (jnp.dot(x_ref[...], w_ref[...], preferred_element_type=jnp.float32)
       + b_ref[...])
  o_ref[0] = h[:, :d_half]
  o_ref[1] = h[:, d_half:]


def _linear_split(x, W, b):
  n, d_in = x.shape
  d_out = W.shape[1]
  blk = 1000
  return pl.pallas_call(
      _matmul_body,
      grid=(n // blk,),
      in_specs=[
          pl.BlockSpec((blk, d_in), lambda i: (i, 0)),
          pl.BlockSpec((d_in, d_out), lambda i: (0, 0)),
          pl.BlockSpec((1, d_out), lambda i: (0, 0)),
      ],
      out_specs=pl.BlockSpec((_NC, blk, d_out // _NC), lambda i: (0, i, 0)),
      out_shape=jax.ShapeDtypeStruct((_NC, n, d_out // _NC), jnp.float32),
  )(x, W, b.reshape(1, d_out))


def _sc_body(n_pad, nb, h_hbm, src_hbm, dst_hbm, adj_hbm, out_hbm,
             sidx, didx, adjv, rows, tmp, acc, gsem):
  c = lax.axis_index("c")
  s = lax.axis_index("s")
  dh = h_hbm.shape[2]
  rows_per_sub = n_pad // _NS
  chunk = 128
  nchunk = rows_per_sub // chunk
  r0 = s * rows_per_sub

  def _zrow(i, _):
    for k in range(dh // _L):
      tmp[i, pl.ds(k * _L, _L)] = jnp.zeros((_L,), jnp.float32)
    return 0
  lax.fori_loop(0, chunk, _zrow, 0)
  for k in range(nchunk):
    pltpu.sync_copy(tmp, acc.at[pl.ds(r0 + k * chunk, chunk)])
  plsc.subcore_barrier()

  def _batch(bidx, _):
    g0 = bidx * _GB
    pltpu.sync_copy(src_hbm.at[s, pl.ds(g0, _GB)], sidx)
    pltpu.sync_copy(dst_hbm.at[s, pl.ds(g0, _GB)], didx)
    pltpu.sync_copy(adj_hbm.at[s, pl.ds(g0, _GB)], adjv)
    cps = [pltpu.async_copy(h_hbm.at[c].at[sidx.at[j]], rows.at[j], gsem)
           for j in range(_GB)]
    for cp in cps:
      cp.wait()
    for j in range(_GB):
      def _scale(t, _):
        a_vec = adjv[j, pl.ds(t * _L, _L)]
        for e in range(_L):
          a = a_vec[e]
          i = t * _L + e
          for k in range(dh // _L):
            rows[j, i, pl.ds(k * _L, _L)] = rows[j, i, pl.ds(k * _L, _L)] * a
        return 0
      lax.fori_loop(0, _GROUP // _L, _scale, 0)
    for j in range(_GB):
      pltpu.sync_copy(rows.at[j], acc.at[didx.at[j]], add=True)
    return 0

  lax.fori_loop(0, nb, _batch, 0)
  plsc.subcore_barrier()

  for k in range(nchunk):
    pltpu.sync_copy(acc.at[pl.ds(r0 + k * chunk, chunk)], tmp)
    pltpu.sync_copy(tmp, out_hbm.at[c, pl.ds(r0 + k * chunk, chunk)])


def _sc_spmm(h_split, src, dst, adj):
  _, n, dh = h_split.shape
  n_pad = ((n + _NS * 128 - 1) // (_NS * 128)) * (_NS * 128)
  e = src.shape[0]
  batch_edges = _GROUP * _GB
  per_w = ((e + _NS * batch_edges - 1) // (_NS * batch_edges)) * batch_edges
  nb = per_w // batch_edges
  e_pad = per_w * _NS
  pad = e_pad - e
  src_p = jnp.concatenate([src, jnp.zeros((pad,), jnp.int32)])
  dst_p = jnp.concatenate([dst, jnp.zeros((pad,), jnp.int32)])
  adj_p = jnp.concatenate([adj, jnp.zeros((pad,), jnp.float32)])
  ng = per_w // _GROUP
  src3 = src_p.reshape(_NS, ng, _GROUP)
  dst3 = dst_p.reshape(_NS, ng, _GROUP)
  adj3 = adj_p.reshape(_NS, ng, _GROUP)

  mesh = plsc.VectorSubcoreMesh(
      core_axis_name="c", subcore_axis_name="s",
      num_cores=_NC, num_subcores=_NS)
  fn = pl.kernel(
      functools.partial(_sc_body, n_pad, nb),
      out_type=jax.ShapeDtypeStruct((_NC, n_pad, dh), jnp.float32),
      mesh=mesh,
      compiler_params=pltpu.CompilerParams(use_tc_tiling_on_sc=False),
      scratch_types=[
          pltpu.VMEM((_GB, _GROUP), jnp.int32),
          pltpu.VMEM((_GB, _GROUP), jnp.int32),
          pltpu.VMEM((_GB, _GROUP), jnp.float32),
          pltpu.VMEM((_GB, _GROUP, dh), jnp.float32),
          pltpu.VMEM((128, dh), jnp.float32),
          pltpu.VMEM_SHARED((n_pad, dh), jnp.float32),
          pltpu.SemaphoreType.DMA,
      ],
  )
  return fn(h_split, src3, dst3, adj3)


def _combine_body(p_ref, o_ref):
  t = jnp.concatenate([p_ref[0], p_ref[1]], axis=1)
  o_ref[...] = jnp.where(t > 0, t, jnp.exp(t) - 1.0)


def _combine(partials, n):
  dh = partials.shape[2]
  blk = 1000
  return pl.pallas_call(
      _combine_body,
      grid=(n // blk,),
      in_specs=[pl.BlockSpec((_NC, blk, dh), lambda i: (0, i, 0))],
      out_specs=pl.BlockSpec((blk, _NC * dh), lambda i: (i, 0)),
      out_shape=jax.ShapeDtypeStruct((n, _NC * dh), jnp.float32),
  )(partials)


def kernel(x, edge_index, adj_values, W, b):
  h_split = _linear_split(x, W, b)
  partials = _sc_spmm(h_split, edge_index[1], edge_index[0], adj_values)
  return _combine(partials, x.shape[0])
